# Initial kernel scaffold; baseline (speedup 1.0000x reference)
import functools

import jax
import jax.numpy as jnp
from jax import lax
from jax.experimental import pallas as pl
from jax.experimental.pallas import tpu as pltpu
from jax.experimental.pallas import tpu_sc as plsc

N = 10000
NPAD = 10240
D_FEAT = 128
HID = 16
NCLS = 16
E = 320000
E_REAL = E + N
K = 128
NC, NS = 2, 16
NW = NC * NS
TPB = 81
E_PAD = NW * TPB * K
R = N // NS

f32 = jnp.float32
i32 = jnp.int32

BN = 2048
GRID = NPAD // BN


def _tc1_body(x_ref, w_ref, asrc_ref, adst_ref, h_ref, as_ref, ad_ref):
    h = x_ref[...] @ w_ref[...]
    h_ref[...] = h
    as_ref[...] = h @ asrc_ref[...]
    ad_ref[...] = h @ adst_ref[...]


_tc1 = pl.pallas_call(
    _tc1_body,
    grid=(GRID,),
    in_specs=[
        pl.BlockSpec((BN, D_FEAT), lambda i: (i, 0)),
        pl.BlockSpec((D_FEAT, HID), lambda i: (0, 0)),
        pl.BlockSpec((HID, 1), lambda i: (0, 0)),
        pl.BlockSpec((HID, 1), lambda i: (0, 0)),
    ],
    out_specs=[
        pl.BlockSpec((BN, HID), lambda i: (i, 0)),
        pl.BlockSpec((BN, 1), lambda i: (i, 0)),
        pl.BlockSpec((BN, 1), lambda i: (i, 0)),
    ],
    out_shape=[
        jax.ShapeDtypeStruct((NPAD, HID), f32),
        jax.ShapeDtypeStruct((NPAD, 1), f32),
        jax.ShapeDtypeStruct((NPAD, 1), f32),
    ],
)


def _tc2_body(num_ref, den_ref, b_ref, w_ref, asrc_ref, adst_ref,
              h_ref, as_ref, ad_ref):
    num = jnp.sum(num_ref[...], axis=0)
    den = jnp.sum(den_ref[...], axis=0)
    xo = num / (den[:, None] + 1e-16) + b_ref[...]
    xo = jnp.maximum(xo, 0.0)
    h = xo @ w_ref[...]
    h_ref[...] = h
    as_ref[...] = h @ asrc_ref[...]
    ad_ref[...] = h @ adst_ref[...]


_tc2 = pl.pallas_call(
    _tc2_body,
    grid=(GRID,),
    in_specs=[
        pl.BlockSpec((NC, BN, HID), lambda i: (0, i, 0)),
        pl.BlockSpec((NW, BN), lambda i: (0, i)),
        pl.BlockSpec((1, HID), lambda i: (0, 0)),
        pl.BlockSpec((HID, HID), lambda i: (0, 0)),
        pl.BlockSpec((HID, 1), lambda i: (0, 0)),
        pl.BlockSpec((HID, 1), lambda i: (0, 0)),
    ],
    out_specs=[
        pl.BlockSpec((BN, HID), lambda i: (i, 0)),
        pl.BlockSpec((BN, 1), lambda i: (i, 0)),
        pl.BlockSpec((BN, 1), lambda i: (i, 0)),
    ],
    out_shape=[
        jax.ShapeDtypeStruct((NPAD, HID), f32),
        jax.ShapeDtypeStruct((NPAD, 1), f32),
        jax.ShapeDtypeStruct((NPAD, 1), f32),
    ],
)


def _tc3_body(num_ref, den_ref, b_ref, w_ref, bo_ref, out_ref):
    num = jnp.sum(num_ref[...], axis=0)
    den = jnp.sum(den_ref[...], axis=0)
    xo = num / (den[:, None] + 1e-16) + b_ref[...]
    xo = jnp.maximum(xo, 0.0)
    logits = xo @ w_ref[...] + bo_ref[...]
    m = jnp.max(logits, axis=1, keepdims=True)
    p = jnp.exp(logits - m)
    out_ref[...] = p / jnp.sum(p, axis=1, keepdims=True)


_tc3 = pl.pallas_call(
    _tc3_body,
    grid=(GRID,),
    in_specs=[
        pl.BlockSpec((NC, BN, HID), lambda i: (0, i, 0)),
        pl.BlockSpec((NW, BN), lambda i: (0, i)),
        pl.BlockSpec((1, HID), lambda i: (0, 0)),
        pl.BlockSpec((HID, NCLS), lambda i: (0, 0)),
        pl.BlockSpec((1, NCLS), lambda i: (0, 0)),
    ],
    out_specs=pl.BlockSpec((BN, NCLS), lambda i: (i, 0)),
    out_shape=jax.ShapeDtypeStruct((NPAD, NCLS), f32),
)


_mesh = plsc.VectorSubcoreMesh(core_axis_name="c", subcore_axis_name="s")


@functools.partial(
    pl.kernel,
    out_type=(
        jax.ShapeDtypeStruct((NC * NPAD, HID), f32),
        jax.ShapeDtypeStruct((NW * NPAD,), f32),
    ),
    mesh=_mesh,
    scratch_types=[
        pltpu.VMEM((N,), f32),
        pltpu.VMEM((N,), f32),
        pltpu.VMEM((N,), f32),
        pltpu.VMEM((TPB, K), i32),
        pltpu.VMEM((TPB, K), i32),
        pltpu.VMEM((K,), f32),
        pltpu.VMEM((K, HID), f32),
        pltpu.VMEM_SHARED((N, HID), f32),
        pltpu.SemaphoreType.DMA,
    ],
)
def _sc_edge(h_hbm, as_hbm, ad_hbm, src_hbm, dst_hbm, num_out, den_out,
             as_l, ad_l, den_l, src_l, dst_l, exb, rows, num_sh, sem):
    c = lax.axis_index("c")
    s = lax.axis_index("s")
    wid = c * NS + s

    pltpu.sync_copy(as_hbm.at[pl.ds(0, N)], as_l)
    pltpu.sync_copy(ad_hbm.at[pl.ds(0, N)], ad_l)
    pltpu.sync_copy(src_hbm.at[pl.ds(wid * TPB, TPB)], src_l)
    pltpu.sync_copy(dst_hbm.at[pl.ds(wid * TPB, TPB)], dst_l)

    zeros16 = jnp.zeros((16,), f32)

    def _zero_den(j, carry):
        den_l[pl.ds(j * 16, 16)] = zeros16
        return carry

    lax.fori_loop(0, N // 16, _zero_den, 0)

    def _zero_rows(j, carry):
        rows[j] = zeros16
        return carry

    lax.fori_loop(0, K, _zero_rows, 0)

    def _zero_num(i, carry):
        pltpu.sync_copy(rows.at[pl.ds(0, R // 5)],
                        num_sh.at[pl.ds(s * R + i * (R // 5), R // 5)])
        return carry

    lax.fori_loop(0, 5, _zero_num, 0)
    plsc.subcore_barrier()

    def _block(b, carry):
        def _grp(g, carry2):
            sl = pl.ds(g * 16, 16)
            s16 = src_l[b, sl]
            d16 = dst_l[b, sl]
            e = plsc.load_gather(as_l, [s16]) + plsc.load_gather(ad_l, [d16])
            e = jnp.maximum(e, e * 0.2)
            ex = jnp.exp(e)
            eid = (wid * TPB + b) * K + g * 16 + lax.iota(i32, 16)
            ex = jnp.where(eid < E_REAL, ex, 0.0)
            plsc.addupdate_scatter(den_l, [d16], ex)
            exb[sl] = ex
            return carry2

        lax.fori_loop(0, K // 16, _grp, 0)

        pltpu.async_copy(h_hbm.at[src_l.at[b]], rows, sem).wait()

        def _scale(j, carry2):
            rows[j] = rows[j] * exb[j]
            return carry2

        lax.fori_loop(0, K, _scale, 0)

        pltpu.sync_copy(rows, num_sh.at[dst_l.at[b]], add=True)
        return carry

    lax.fori_loop(0, TPB, _block, 0)
    plsc.subcore_barrier()

    pltpu.sync_copy(den_l, den_out.at[pl.ds(wid * NPAD, N)])
    pltpu.sync_copy(num_sh.at[pl.ds(s * R, R)],
                    num_out.at[pl.ds(c * NPAD + s * R, R)])


def kernel(x, edge_index, W1, att_src1, att_dst1, b1,
           W2, att_src2, att_dst2, b2, Wout, bout):
    loop = jnp.arange(N, dtype=i32)
    padi = jnp.zeros((E_PAD - E_REAL,), dtype=i32)
    src = jnp.concatenate([edge_index[0].astype(i32), loop, padi])
    dst = jnp.concatenate([edge_index[1].astype(i32), loop, padi])
    src2d = src.reshape(NW * TPB, K)
    dst2d = dst.reshape(NW * TPB, K)

    xp = jnp.pad(x, ((0, NPAD - N), (0, 0)))

    h1, as1, ad1 = _tc1(xp, W1, att_src1.reshape(HID, 1),
                        att_dst1.reshape(HID, 1))
    num1, den1 = _sc_edge(h1, as1.reshape(NPAD), ad1.reshape(NPAD),
                          src2d, dst2d)
    h2, as2, ad2 = _tc2(num1.reshape(NC, NPAD, HID),
                        den1.reshape(NW, NPAD),
                        b1.reshape(1, HID), W2,
                        att_src2.reshape(HID, 1), att_dst2.reshape(HID, 1))
    num2, den2 = _sc_edge(h2, as2.reshape(NPAD), ad2.reshape(NPAD),
                          src2d, dst2d)
    probs = _tc3(num2.reshape(NC, NPAD, HID),
                 den2.reshape(NW, NPAD),
                 b2.reshape(1, HID), Wout, bout.reshape(1, NCLS))
    return probs[:N]


# speedup vs baseline: 61.5412x; 61.5412x over previous
import functools

import jax
import jax.numpy as jnp
from jax import lax
from jax.experimental import pallas as pl
from jax.experimental.pallas import tpu as pltpu
from jax.experimental.pallas import tpu_sc as plsc

N = 10000
NPAD = 10240
D_FEAT = 128
HID = 16
NCLS = 16
E = 320000
E_REAL = E + N
K = 128
NC, NS = 2, 16
NW = NC * NS
TPB = 81
E_PAD = NW * TPB * K
R = NPAD // NS

f32 = jnp.float32
i32 = jnp.int32

BN = 2048
GRID = NPAD // BN


def _tc1_body(x_ref, w_ref, asrc_ref, adst_ref, h_ref, as_ref, ad_ref):
    h = x_ref[...] @ w_ref[...]
    h_ref[...] = h
    as_ref[...] = h @ asrc_ref[...]
    ad_ref[...] = h @ adst_ref[...]


_tc1 = pl.pallas_call(
    _tc1_body,
    grid=(GRID,),
    in_specs=[
        pl.BlockSpec((BN, D_FEAT), lambda i: (i, 0)),
        pl.BlockSpec((D_FEAT, HID), lambda i: (0, 0)),
        pl.BlockSpec((HID, 1), lambda i: (0, 0)),
        pl.BlockSpec((HID, 1), lambda i: (0, 0)),
    ],
    out_specs=[
        pl.BlockSpec((BN, HID), lambda i: (i, 0)),
        pl.BlockSpec((BN, 1), lambda i: (i, 0)),
        pl.BlockSpec((BN, 1), lambda i: (i, 0)),
    ],
    out_shape=[
        jax.ShapeDtypeStruct((NPAD, HID), f32),
        jax.ShapeDtypeStruct((NPAD, 1), f32),
        jax.ShapeDtypeStruct((NPAD, 1), f32),
    ],
)


def _tc2_body(num_ref, den_ref, b_ref, w_ref, asrc_ref, adst_ref,
              h_ref, as_ref, ad_ref):
    num = jnp.sum(num_ref[...], axis=0)
    den = jnp.sum(den_ref[...], axis=0)
    xo = num / (den[:, None] + 1e-16) + b_ref[...]
    xo = jnp.maximum(xo, 0.0)
    h = xo @ w_ref[...]
    h_ref[...] = h
    as_ref[...] = h @ asrc_ref[...]
    ad_ref[...] = h @ adst_ref[...]


_tc2 = pl.pallas_call(
    _tc2_body,
    grid=(GRID,),
    in_specs=[
        pl.BlockSpec((NC, BN, HID), lambda i: (0, i, 0)),
        pl.BlockSpec((NW, BN), lambda i: (0, i)),
        pl.BlockSpec((1, HID), lambda i: (0, 0)),
        pl.BlockSpec((HID, HID), lambda i: (0, 0)),
        pl.BlockSpec((HID, 1), lambda i: (0, 0)),
        pl.BlockSpec((HID, 1), lambda i: (0, 0)),
    ],
    out_specs=[
        pl.BlockSpec((BN, HID), lambda i: (i, 0)),
        pl.BlockSpec((BN, 1), lambda i: (i, 0)),
        pl.BlockSpec((BN, 1), lambda i: (i, 0)),
    ],
    out_shape=[
        jax.ShapeDtypeStruct((NPAD, HID), f32),
        jax.ShapeDtypeStruct((NPAD, 1), f32),
        jax.ShapeDtypeStruct((NPAD, 1), f32),
    ],
)


def _tc3_body(num_ref, den_ref, b_ref, w_ref, bo_ref, out_ref):
    num = jnp.sum(num_ref[...], axis=0)
    den = jnp.sum(den_ref[...], axis=0)
    xo = num / (den[:, None] + 1e-16) + b_ref[...]
    xo = jnp.maximum(xo, 0.0)
    logits = xo @ w_ref[...] + bo_ref[...]
    m = jnp.max(logits, axis=1, keepdims=True)
    p = jnp.exp(logits - m)
    out_ref[...] = p / jnp.sum(p, axis=1, keepdims=True)


_tc3 = pl.pallas_call(
    _tc3_body,
    grid=(GRID,),
    in_specs=[
        pl.BlockSpec((NC, BN, HID), lambda i: (0, i, 0)),
        pl.BlockSpec((NW, BN), lambda i: (0, i)),
        pl.BlockSpec((1, HID), lambda i: (0, 0)),
        pl.BlockSpec((HID, NCLS), lambda i: (0, 0)),
        pl.BlockSpec((1, NCLS), lambda i: (0, 0)),
    ],
    out_specs=pl.BlockSpec((BN, NCLS), lambda i: (i, 0)),
    out_shape=jax.ShapeDtypeStruct((NPAD, NCLS), f32),
)


_mesh = plsc.VectorSubcoreMesh(core_axis_name="c", subcore_axis_name="s")


@functools.partial(
    pl.kernel,
    out_type=(
        jax.ShapeDtypeStruct((NC * NPAD, HID), f32),
        jax.ShapeDtypeStruct((NW * NPAD,), f32),
    ),
    mesh=_mesh,
    compiler_params=pltpu.CompilerParams(needs_layout_passes=False,
                                         use_tc_tiling_on_sc=False),
    scratch_types=[
        pltpu.VMEM((N,), f32),
        pltpu.VMEM((N,), f32),
        pltpu.VMEM((N,), f32),
        pltpu.VMEM((TPB, K), i32),
        pltpu.VMEM((TPB, K), i32),
        pltpu.VMEM((K,), f32),
        pltpu.VMEM((K, HID), f32),
        pltpu.VMEM_SHARED((NPAD, HID), f32),
        pltpu.SemaphoreType.DMA,
    ],
)
def _sc_edge(h_hbm, as_hbm, ad_hbm, src_hbm, dst_hbm, num_out, den_out,
             as_l, ad_l, den_l, src_l, dst_l, exb, rows, num_sh, sem):
    c = lax.axis_index("c")
    s = lax.axis_index("s")
    wid = c * NS + s

    pltpu.sync_copy(as_hbm.at[pl.ds(0, N)], as_l)
    pltpu.sync_copy(ad_hbm.at[pl.ds(0, N)], ad_l)
    pltpu.sync_copy(src_hbm.at[wid], src_l)
    pltpu.sync_copy(dst_hbm.at[wid], dst_l)

    zeros16 = jnp.zeros((16,), f32)

    def _zero_den(j, carry):
        den_l[pl.ds(j * 16, 16)] = zeros16
        return carry

    lax.fori_loop(0, N // 16, _zero_den, 0)

    def _zero_rows(j, carry):
        rows[j] = zeros16
        return carry

    lax.fori_loop(0, K, _zero_rows, 0)

    def _zero_num(i, carry):
        pltpu.sync_copy(rows, num_sh.at[pl.ds(s * R + i * K, K)])
        return carry

    lax.fori_loop(0, R // K, _zero_num, 0)
    plsc.subcore_barrier()

    def _block(b, carry):
        def _grp(g, carry2):
            sl = pl.ds(g * 16, 16)
            s16 = src_l[b, sl]
            d16 = dst_l[b, sl]
            e = plsc.load_gather(as_l, [s16]) + plsc.load_gather(ad_l, [d16])
            e = jnp.maximum(e, e * 0.2)
            ex = jnp.exp(e)
            eid = (wid * TPB + b) * K + g * 16 + lax.iota(i32, 16)
            ex = jnp.where(eid < E_REAL, ex, 0.0)
            plsc.addupdate_scatter(den_l, [d16], ex)
            exb[sl] = ex
            return carry2

        lax.fori_loop(0, K // 16, _grp, 0)

        pltpu.async_copy(h_hbm.at[src_l.at[b]], rows, sem).wait()

        def _scale(g, carry2):
            exv = exb[pl.ds(g * 16, 16)]
            for j in range(16):
                r = g * 16 + j
                rows[r] = rows[r] * exv[j]
            return carry2

        lax.fori_loop(0, K // 16, _scale, 0)

        pltpu.sync_copy(rows, num_sh.at[dst_l.at[b]], add=True)
        return carry

    lax.fori_loop(0, TPB, _block, 0)
    plsc.subcore_barrier()

    pltpu.sync_copy(den_l, den_out.at[pl.ds(wid * NPAD, N)])
    pltpu.sync_copy(num_sh.at[pl.ds(s * R, R)],
                    num_out.at[pl.ds(c * NPAD + s * R, R)])


def kernel(x, edge_index, W1, att_src1, att_dst1, b1,
           W2, att_src2, att_dst2, b2, Wout, bout):
    loop = jnp.arange(N, dtype=i32)
    padi = jnp.zeros((E_PAD - E_REAL,), dtype=i32)
    src = jnp.concatenate([edge_index[0].astype(i32), loop, padi])
    dst = jnp.concatenate([edge_index[1].astype(i32), loop, padi])
    src2d = src.reshape(NW, TPB, K)
    dst2d = dst.reshape(NW, TPB, K)

    xp = jnp.pad(x, ((0, NPAD - N), (0, 0)))

    h1, as1, ad1 = _tc1(xp, W1, att_src1.reshape(HID, 1),
                        att_dst1.reshape(HID, 1))
    num1, den1 = _sc_edge(h1, as1.reshape(NPAD), ad1.reshape(NPAD),
                          src2d, dst2d)
    h2, as2, ad2 = _tc2(num1.reshape(NC, NPAD, HID),
                        den1.reshape(NW, NPAD),
                        b1.reshape(1, HID), W2,
                        att_src2.reshape(HID, 1), att_dst2.reshape(HID, 1))
    num2, den2 = _sc_edge(h2, as2.reshape(NPAD), ad2.reshape(NPAD),
                          src2d, dst2d)
    probs = _tc3(num2.reshape(NC, NPAD, HID),
                 den2.reshape(NW, NPAD),
                 b2.reshape(1, HID), Wout, bout.reshape(1, NCLS))
    return probs[:N]


# speedup vs baseline: 81.3813x; 1.3224x over previous
import functools

import jax
import jax.numpy as jnp
from jax import lax
from jax.experimental import pallas as pl
from jax.experimental.pallas import tpu as pltpu
from jax.experimental.pallas import tpu_sc as plsc

N = 10000
NPAD = 10240
D_FEAT = 128
HID = 16
NCLS = 16
E = 320000
E_REAL = E + N
K = 128
NC, NS = 2, 16
NW = NC * NS
TPB = 82
E_PAD = NW * TPB * K
R = NPAD // NS

f32 = jnp.float32
i32 = jnp.int32

BN = 2048
GRID = NPAD // BN


def _tc1_body(x_ref, w_ref, asrc_ref, adst_ref, h_ref, as_ref, ad_ref):
    h = x_ref[...] @ w_ref[...]
    h_ref[...] = h
    as_ref[...] = h @ asrc_ref[...]
    ad_ref[...] = h @ adst_ref[...]


_tc1 = pl.pallas_call(
    _tc1_body,
    grid=(GRID,),
    in_specs=[
        pl.BlockSpec((BN, D_FEAT), lambda i: (i, 0)),
        pl.BlockSpec((D_FEAT, HID), lambda i: (0, 0)),
        pl.BlockSpec((HID, 1), lambda i: (0, 0)),
        pl.BlockSpec((HID, 1), lambda i: (0, 0)),
    ],
    out_specs=[
        pl.BlockSpec((BN, HID), lambda i: (i, 0)),
        pl.BlockSpec((BN, 1), lambda i: (i, 0)),
        pl.BlockSpec((BN, 1), lambda i: (i, 0)),
    ],
    out_shape=[
        jax.ShapeDtypeStruct((NPAD, HID), f32),
        jax.ShapeDtypeStruct((NPAD, 1), f32),
        jax.ShapeDtypeStruct((NPAD, 1), f32),
    ],
)


def _tc2_body(num_ref, den_ref, b_ref, w_ref, asrc_ref, adst_ref,
              h_ref, as_ref, ad_ref):
    num = jnp.sum(num_ref[...], axis=0)
    den = jnp.sum(den_ref[...], axis=0)
    xo = num / (den[:, None] + 1e-16) + b_ref[...]
    xo = jnp.maximum(xo, 0.0)
    h = xo @ w_ref[...]
    h_ref[...] = h
    as_ref[...] = h @ asrc_ref[...]
    ad_ref[...] = h @ adst_ref[...]


_tc2 = pl.pallas_call(
    _tc2_body,
    grid=(GRID,),
    in_specs=[
        pl.BlockSpec((NC, BN, HID), lambda i: (0, i, 0)),
        pl.BlockSpec((NW, BN), lambda i: (0, i)),
        pl.BlockSpec((1, HID), lambda i: (0, 0)),
        pl.BlockSpec((HID, HID), lambda i: (0, 0)),
        pl.BlockSpec((HID, 1), lambda i: (0, 0)),
        pl.BlockSpec((HID, 1), lambda i: (0, 0)),
    ],
    out_specs=[
        pl.BlockSpec((BN, HID), lambda i: (i, 0)),
        pl.BlockSpec((BN, 1), lambda i: (i, 0)),
        pl.BlockSpec((BN, 1), lambda i: (i, 0)),
    ],
    out_shape=[
        jax.ShapeDtypeStruct((NPAD, HID), f32),
        jax.ShapeDtypeStruct((NPAD, 1), f32),
        jax.ShapeDtypeStruct((NPAD, 1), f32),
    ],
)


def _tc3_body(num_ref, den_ref, b_ref, w_ref, bo_ref, out_ref):
    num = jnp.sum(num_ref[...], axis=0)
    den = jnp.sum(den_ref[...], axis=0)
    xo = num / (den[:, None] + 1e-16) + b_ref[...]
    xo = jnp.maximum(xo, 0.0)
    logits = xo @ w_ref[...] + bo_ref[...]
    m = jnp.max(logits, axis=1, keepdims=True)
    p = jnp.exp(logits - m)
    out_ref[...] = p / jnp.sum(p, axis=1, keepdims=True)


_tc3 = pl.pallas_call(
    _tc3_body,
    grid=(GRID,),
    in_specs=[
        pl.BlockSpec((NC, BN, HID), lambda i: (0, i, 0)),
        pl.BlockSpec((NW, BN), lambda i: (0, i)),
        pl.BlockSpec((1, HID), lambda i: (0, 0)),
        pl.BlockSpec((HID, NCLS), lambda i: (0, 0)),
        pl.BlockSpec((1, NCLS), lambda i: (0, 0)),
    ],
    out_specs=pl.BlockSpec((BN, NCLS), lambda i: (i, 0)),
    out_shape=jax.ShapeDtypeStruct((NPAD, NCLS), f32),
)


_mesh = plsc.VectorSubcoreMesh(core_axis_name="c", subcore_axis_name="s")


@functools.partial(
    pl.kernel,
    out_type=(
        jax.ShapeDtypeStruct((NC * NPAD, HID), f32),
        jax.ShapeDtypeStruct((NW * NPAD,), f32),
    ),
    mesh=_mesh,
    compiler_params=pltpu.CompilerParams(needs_layout_passes=False,
                                         use_tc_tiling_on_sc=False),
    scratch_types=[
        pltpu.VMEM((N,), f32),
        pltpu.VMEM((N,), f32),
        pltpu.VMEM((N,), f32),
        pltpu.VMEM((TPB, K), i32),
        pltpu.VMEM((TPB, K), i32),
        pltpu.VMEM((K,), f32),
        pltpu.VMEM((K, HID), f32),
        pltpu.VMEM((K, HID), f32),
        pltpu.VMEM_SHARED((NPAD, HID), f32),
        pltpu.SemaphoreType.DMA,
        pltpu.SemaphoreType.DMA,
        pltpu.SemaphoreType.DMA,
        pltpu.SemaphoreType.DMA,
    ],
)
def _sc_edge(h_hbm, as_hbm, ad_hbm, src_hbm, dst_hbm, num_out, den_out,
             as_l, ad_l, den_l, src_l, dst_l, exb, rows0, rows1, num_sh,
             sem_g0, sem_g1, sem_s0, sem_s1):
    c = lax.axis_index("c")
    s = lax.axis_index("s")
    wid = c * NS + s

    pltpu.sync_copy(as_hbm.at[pl.ds(0, N)], as_l)
    pltpu.sync_copy(ad_hbm.at[pl.ds(0, N)], ad_l)
    pltpu.sync_copy(src_hbm.at[wid], src_l)
    pltpu.sync_copy(dst_hbm.at[wid], dst_l)

    zeros16 = jnp.zeros((16,), f32)

    def _zero_den(j, carry):
        den_l[pl.ds(j * 16, 16)] = zeros16
        return carry

    lax.fori_loop(0, N // 16, _zero_den, 0)

    def _zero_rows(j, carry):
        rows0[j] = zeros16
        return carry

    lax.fori_loop(0, K, _zero_rows, 0)

    def _zero_num(i, carry):
        pltpu.sync_copy(rows0, num_sh.at[pl.ds(s * R + i * K, K)])
        return carry

    lax.fori_loop(0, R // K, _zero_num, 0)
    plsc.subcore_barrier()

    def _phase1(b):
        def _grp(g, carry2):
            sl = pl.ds(g * 16, 16)
            s16 = src_l[b, sl]
            d16 = dst_l[b, sl]
            e = plsc.load_gather(as_l, [s16]) + plsc.load_gather(ad_l, [d16])
            e = jnp.maximum(e, e * 0.2)
            ex = jnp.exp(e)
            eid = (wid * TPB + b) * K + g * 16 + lax.iota(i32, 16)
            ex = jnp.where(eid < E_REAL, ex, 0.0)
            plsc.addupdate_scatter(den_l, [d16], ex)
            exb[sl] = ex
            return carry2

        lax.fori_loop(0, K // 16, _grp, 0)

    def _scale(buf):
        def _s(g, carry2):
            exv = exb[pl.ds(g * 16, 16)]
            for j in range(16):
                r = g * 16 + j
                buf[r] = buf[r] * exv[j]
            return carry2

        lax.fori_loop(0, K // 16, _s, 0)

    S = TPB // 2
    pltpu.async_copy(h_hbm.at[src_l.at[0]], rows0, sem_g0)

    def _super(i, carry):
        b0 = 2 * i
        b1 = b0 + 1
        _phase1(b0)

        @pl.when(i > 0)
        def _():
            pltpu.make_async_copy(
                rows1, num_sh.at[dst_l.at[b0 - 1]], sem_s1).wait()

        pltpu.async_copy(h_hbm.at[src_l.at[b1]], rows1, sem_g1)
        pltpu.make_async_copy(h_hbm.at[src_l.at[b0]], rows0, sem_g0).wait()
        _scale(rows0)
        pltpu.async_copy(rows0, num_sh.at[dst_l.at[b0]], sem_s0, add=True)
        _phase1(b1)

        @pl.when(i < S - 1)
        def _():
            pltpu.make_async_copy(
                rows0, num_sh.at[dst_l.at[b0]], sem_s0).wait()
            pltpu.async_copy(h_hbm.at[src_l.at[b0 + 2]], rows0, sem_g0)

        pltpu.make_async_copy(h_hbm.at[src_l.at[b1]], rows1, sem_g1).wait()
        _scale(rows1)
        pltpu.async_copy(rows1, num_sh.at[dst_l.at[b1]], sem_s1, add=True)
        return carry

    lax.fori_loop(0, S, _super, 0)
    pltpu.make_async_copy(rows0, num_sh.at[dst_l.at[TPB - 2]], sem_s0).wait()
    pltpu.make_async_copy(rows1, num_sh.at[dst_l.at[TPB - 1]], sem_s1).wait()
    plsc.subcore_barrier()

    pltpu.sync_copy(den_l, den_out.at[pl.ds(wid * NPAD, N)])
    pltpu.sync_copy(num_sh.at[pl.ds(s * R, R)],
                    num_out.at[pl.ds(c * NPAD + s * R, R)])


def kernel(x, edge_index, W1, att_src1, att_dst1, b1,
           W2, att_src2, att_dst2, b2, Wout, bout):
    loop = jnp.arange(N, dtype=i32)
    padi = jnp.zeros((E_PAD - E_REAL,), dtype=i32)
    src = jnp.concatenate([edge_index[0].astype(i32), loop, padi])
    dst = jnp.concatenate([edge_index[1].astype(i32), loop, padi])
    src2d = src.reshape(NW, TPB, K)
    dst2d = dst.reshape(NW, TPB, K)

    xp = jnp.pad(x, ((0, NPAD - N), (0, 0)))

    h1, as1, ad1 = _tc1(xp, W1, att_src1.reshape(HID, 1),
                        att_dst1.reshape(HID, 1))
    num1, den1 = _sc_edge(h1, as1.reshape(NPAD), ad1.reshape(NPAD),
                          src2d, dst2d)
    h2, as2, ad2 = _tc2(num1.reshape(NC, NPAD, HID),
                        den1.reshape(NW, NPAD),
                        b1.reshape(1, HID), W2,
                        att_src2.reshape(HID, 1), att_dst2.reshape(HID, 1))
    num2, den2 = _sc_edge(h2, as2.reshape(NPAD), ad2.reshape(NPAD),
                          src2d, dst2d)
    probs = _tc3(num2.reshape(NC, NPAD, HID),
                 den2.reshape(NW, NPAD),
                 b2.reshape(1, HID), Wout, bout.reshape(1, NCLS))
    return probs[:N]
